# Initial kernel scaffold; baseline (speedup 1.0000x reference)
import jax
import jax.numpy as jnp
from jax.experimental import pallas as pl


def kernel(x, adj_indices, adj_values, W1, b1, bn1_gamma, bn1_beta, W2, b2, bn2_gamma, bn2_beta, fc1_W, fc1_b, fc2_W, fc2_b):
    raise NotImplementedError("write your pallas kernel here")


# speedup vs baseline: 6.9346x; 6.9346x over previous
import functools

import jax
import jax.numpy as jnp
from jax import lax
from jax.experimental import pallas as pl
from jax.experimental.pallas import tpu as pltpu
from jax.experimental.pallas import tpu_sc as plsc

N = 10000
E = 320000
F = 128
EPS = 1e-5

NC = 2
NS = 16
NW = NC * NS
L = 16

NPAD = 10240
NCHK = NPAD // NS

CH = 128
NCHUNK = 80
EPW = NCHUNK * CH
EP = NW * EPW

ISQ = float(1.0 / (1.0 + EPS) ** 0.5)


@functools.lru_cache(maxsize=1)
def _sc_kernels():
    mesh = plsc.VectorSubcoreMesh(
        core_axis_name="c", subcore_axis_name="s",
        num_cores=NC, num_subcores=NS,
    )

    @functools.partial(
        pl.kernel,
        out_type=jax.ShapeDtypeStruct((NC, NPAD), jnp.float32),
        mesh=mesh,
        compiler_params=pltpu.CompilerParams(needs_layout_passes=False),
        scratch_types=[
            pltpu.VMEM((NCHUNK, CH), jnp.int32),
            pltpu.VMEM((NCHUNK, CH), jnp.float32),
            pltpu.VMEM((NCHK,), jnp.float32),
            pltpu.VMEM_SHARED((NPAD,), jnp.float32),
        ],
    )
    def deg_kernel(col_b, ew_b, deg_out, eidx, evalf, tmp, deg_sh):
        cid = lax.axis_index("c")
        sid = lax.axis_index("s")
        wid = sid * NC + cid
        nbase = sid * NCHK

        def _z(i, c):
            tmp[pl.ds(i * L, L)] = jnp.zeros((L,), jnp.float32)
            return c
        lax.fori_loop(0, NCHK // L, _z, 0)
        pltpu.sync_copy(tmp, deg_sh.at[pl.ds(nbase, NCHK)])
        plsc.subcore_barrier()

        pltpu.sync_copy(col_b.at[wid], eidx)
        pltpu.sync_copy(ew_b.at[wid], evalf)

        def _deg(j, c):
            pltpu.sync_copy(evalf.at[j], deg_sh.at[eidx.at[j]], add=True)
            return c
        lax.fori_loop(0, NCHUNK, _deg, 0)
        plsc.subcore_barrier()
        pltpu.sync_copy(deg_sh.at[pl.ds(nbase, NCHK)],
                        deg_out.at[cid, pl.ds(nbase, NCHK)])

    @functools.partial(
        pl.kernel,
        out_type=jax.ShapeDtypeStruct((NW, NCHUNK, CH), jnp.float32),
        mesh=mesh,
        compiler_params=pltpu.CompilerParams(needs_layout_passes=False),
        scratch_types=[
            pltpu.VMEM((NCHUNK, CH), jnp.int32),
            pltpu.VMEM((NCHUNK, CH), jnp.int32),
            pltpu.VMEM((NCHUNK, CH), jnp.float32),
            pltpu.VMEM((NCHUNK, CH), jnp.float32),
            pltpu.VMEM((NPAD,), jnp.float32),
        ],
    )
    def norm_kernel(row_b, col_b, ew_b, dinv, norm_out,
                    ridx, cidx, ewb, nbuf, dinv_loc):
        cid = lax.axis_index("c")
        sid = lax.axis_index("s")
        wid = sid * NC + cid

        pltpu.sync_copy(dinv, dinv_loc)
        pltpu.sync_copy(row_b.at[wid], ridx)
        pltpu.sync_copy(col_b.at[wid], cidx)
        pltpu.sync_copy(ew_b.at[wid], ewb)

        def _nrm(j, c):
            for k in range(CH // L):
                sl = pl.ds(k * L, L)
                ri = ridx[j, sl]
                ci = cidx[j, sl]
                w = ewb[j, sl]
                dv = plsc.load_gather(dinv_loc, [ri])
                dc = plsc.load_gather(dinv_loc, [ci])
                nbuf[j, sl] = dv * w * dc
            return c
        lax.fori_loop(0, NCHUNK, _nrm, 0)
        pltpu.sync_copy(nbuf, norm_out.at[wid])

    @functools.partial(
        pl.kernel,
        out_type=jax.ShapeDtypeStruct((NC, NPAD, F), jnp.float32),
        mesh=mesh,
        compiler_params=pltpu.CompilerParams(needs_layout_passes=False),
        scratch_types=[
            pltpu.VMEM((NCHUNK, CH), jnp.int32),
            pltpu.VMEM((NCHUNK, CH), jnp.int32),
            pltpu.VMEM((NCHUNK, CH), jnp.float32),
            pltpu.VMEM((CH, F), jnp.float32),
            pltpu.VMEM_SHARED((NPAD, F), jnp.float32),
            pltpu.SemaphoreType.DMA,
        ],
    )
    def scatter_kernel(h_pad, row_b, col_b, norm_b, part,
                       ridx, cidx, nrm, rows, acc_sh, sem):
        cid = lax.axis_index("c")
        sid = lax.axis_index("s")
        wid = sid * NC + cid
        nbase = sid * NCHK

        def _zr(r, c):
            for k in range(F // L):
                rows[r, pl.ds(k * L, L)] = jnp.zeros((L,), jnp.float32)
            return c
        lax.fori_loop(0, CH, _zr, 0)
        for b in range(NCHK // CH):
            pltpu.sync_copy(rows, acc_sh.at[pl.ds(nbase + b * CH, CH)])
        plsc.subcore_barrier()

        pltpu.sync_copy(row_b.at[wid], ridx)
        pltpu.sync_copy(col_b.at[wid], cidx)
        pltpu.sync_copy(norm_b.at[wid], nrm)

        def _edge(j, c):
            pltpu.async_copy(h_pad.at[ridx.at[j]], rows, sem).wait()

            def _sce(e, c2):
                je = jnp.full((L,), j, jnp.int32)
                ee = jnp.full((L,), e, jnp.int32)
                s = plsc.load_gather(nrm, [je, ee])
                for k in range(F // L):
                    sl = pl.ds(k * L, L)
                    rows[e, sl] = rows[e, sl] * s
                return c2
            lax.fori_loop(0, CH, _sce, 0)
            pltpu.sync_copy(rows, acc_sh.at[cidx.at[j]], add=True)
            return c
        lax.fori_loop(0, NCHUNK, _edge, 0)
        plsc.subcore_barrier()

        for b in range(NCHK // CH):
            sl = pl.ds(nbase + b * CH, CH)
            pltpu.sync_copy(acc_sh.at[sl], part.at[cid, sl])

    return deg_kernel, norm_kernel, scatter_kernel


_BLK = 1000


def _dinv_body(dp_ref, dinv_ref, sn_ref):
    d = dp_ref[0] + dp_ref[1] + 1.0
    r = lax.rsqrt(d)
    dinv_ref[...] = r
    sn_ref[...] = r * r


def _tc_dinv(deg_parts):
    m = NPAD // F
    return pl.pallas_call(
        _dinv_body,
        out_shape=(
            jax.ShapeDtypeStruct((m, F), jnp.float32),
            jax.ShapeDtypeStruct((m, F), jnp.float32),
        ),
    )(deg_parts)


def _mm_body(x_ref, w_ref, o_ref):
    o_ref[...] = jnp.dot(x_ref[...], w_ref[...],
                         preferred_element_type=jnp.float32)


def _tc_matmul(x, w):
    return pl.pallas_call(
        _mm_body,
        grid=(N // _BLK,),
        in_specs=[
            pl.BlockSpec((_BLK, F), lambda i: (i, 0)),
            pl.BlockSpec((F, F), lambda i: (0, 0)),
        ],
        out_specs=pl.BlockSpec((_BLK, F), lambda i: (i, 0)),
        out_shape=jax.ShapeDtypeStruct((N, F), jnp.float32),
    )(x, w)


def _comb_body(p0_ref, p1_ref, hm_ref, sn_ref, b_ref, g_ref, bt_ref, w2_ref,
               h1_ref, h2m_ref):
    s = p0_ref[...] + p1_ref[...] + hm_ref[...] * sn_ref[...]
    h = jnp.maximum(s + b_ref[...], 0.0)
    h = h * (g_ref[...] * ISQ) + bt_ref[...]
    h1_ref[...] = h
    h2m_ref[...] = jnp.dot(h, w2_ref[...], preferred_element_type=jnp.float32)


def _tc_combine_mm(p0, p1, hm, sn, b, g, bt, w2):
    vec = pl.BlockSpec((1, F), lambda i: (0, 0))
    blk = pl.BlockSpec((_BLK, F), lambda i: (i, 0))
    col = pl.BlockSpec((_BLK, 1), lambda i: (i, 0))
    return pl.pallas_call(
        _comb_body,
        grid=(N // _BLK,),
        in_specs=[blk, blk, blk, col, vec, vec, vec,
                  pl.BlockSpec((F, F), lambda i: (0, 0))],
        out_specs=(blk, blk),
        out_shape=(
            jax.ShapeDtypeStruct((N, F), jnp.float32),
            jax.ShapeDtypeStruct((N, F), jnp.float32),
        ),
    )(p0, p1, hm, sn, b.reshape(1, F), g.reshape(1, F), bt.reshape(1, F), w2)


def _head_body(p0_ref, p1_ref, hm_ref, sn_ref, b_ref, g_ref, bt_ref,
               x_ref, h1_ref,
               f1a_ref, f1b_ref, f1c_ref, fb1_ref, f2w_ref, f2b_ref, o_ref):
    s = p0_ref[...] + p1_ref[...] + hm_ref[...] * sn_ref[...]
    h2 = jnp.maximum(s + b_ref[...], 0.0)
    h2 = h2 * (g_ref[...] * ISQ) + bt_ref[...]
    z = (jnp.dot(x_ref[...], f1a_ref[...], preferred_element_type=jnp.float32)
         + jnp.dot(h1_ref[...], f1b_ref[...],
                   preferred_element_type=jnp.float32)
         + jnp.dot(h2, f1c_ref[...], preferred_element_type=jnp.float32))
    z = jnp.maximum(z + fb1_ref[...], 0.0)
    y = jnp.dot(z, f2w_ref[...], preferred_element_type=jnp.float32)
    o_ref[...] = jnp.maximum(y + f2b_ref[...], 0.0)


def _tc_head(p0, p1, hm, sn, b2, g2, bt2, x, h1, f1a, f1b, f1c, fb1,
             f2w, f2b):
    vec = pl.BlockSpec((1, F), lambda i: (0, 0))
    blk = pl.BlockSpec((_BLK, F), lambda i: (i, 0))
    col = pl.BlockSpec((_BLK, 1), lambda i: (i, 0))
    wspec = pl.BlockSpec((F, F), lambda i: (0, 0))
    return pl.pallas_call(
        _head_body,
        grid=(N // _BLK,),
        in_specs=[blk, blk, blk, col, vec, vec, vec, blk, blk,
                  wspec, wspec, wspec,
                  vec, pl.BlockSpec((F, 1), lambda i: (0, 0)),
                  pl.BlockSpec((1, 1), lambda i: (0, 0))],
        out_specs=pl.BlockSpec((_BLK, 1), lambda i: (i, 0)),
        out_shape=jax.ShapeDtypeStruct((N, 1), jnp.float32),
    )(p0, p1, hm, sn, b2.reshape(1, F), g2.reshape(1, F), bt2.reshape(1, F),
      x, h1, f1a, f1b, f1c, fb1.reshape(1, F), f2w, f2b.reshape(1, 1))


def kernel(x, adj_indices, adj_values, W1, b1, bn1_gamma, bn1_beta,
           W2, b2, bn2_gamma, bn2_beta, fc1_W, fc1_b, fc2_W, fc2_b):
    deg_kernel, norm_kernel, scatter_kernel = _sc_kernels()
    ipad = jnp.zeros((EP - E,), jnp.int32)
    row_b = jnp.concatenate([adj_indices[0], ipad]).reshape(NW, NCHUNK, CH)
    col_b = jnp.concatenate([adj_indices[1], ipad]).reshape(NW, NCHUNK, CH)
    ew_b = jnp.concatenate([adj_values, jnp.zeros((EP - E,), jnp.float32)]
                           ).reshape(NW, NCHUNK, CH)

    deg_parts = deg_kernel(col_b, ew_b)
    dinv2d, sn2d = _tc_dinv(deg_parts.reshape(NC, NPAD // F, F))
    norm_b = norm_kernel(row_b, col_b, ew_b, dinv2d.reshape(NPAD))
    sn = sn2d.reshape(NPAD)[:N].reshape(N, 1)

    h1m = _tc_matmul(x, W1)
    part1 = scatter_kernel(h1m, row_b, col_b, norm_b)
    h1, h2m = _tc_combine_mm(part1[0, :N], part1[1, :N], h1m, sn,
                             b1, bn1_gamma, bn1_beta, W2)
    part2 = scatter_kernel(h2m, row_b, col_b, norm_b)
    y = _tc_head(part2[0, :N], part2[1, :N], h2m, sn, b2, bn2_gamma, bn2_beta,
                 x, h1, fc1_W[:F], fc1_W[F:2 * F], fc1_W[2 * F:],
                 fc1_b, fc2_W, fc2_b)
    return y.reshape(-1)


# speedup vs baseline: 8.7110x; 1.2562x over previous
import functools

import jax
import jax.numpy as jnp
from jax import lax
from jax.experimental import pallas as pl
from jax.experimental.pallas import tpu as pltpu
from jax.experimental.pallas import tpu_sc as plsc

N = 10000
E = 320000
F = 128
EPS = 1e-5

NC = 2
NS = 16
NW = NC * NS
L = 16

NPAD = 10240
NCHK = NPAD // NS

CH = 128
NCHUNK = 80
EPW = NCHUNK * CH
EP = NW * EPW

ISQ = float(1.0 / (1.0 + EPS) ** 0.5)


@functools.lru_cache(maxsize=1)
def _sc_kernels():
    mesh = plsc.VectorSubcoreMesh(
        core_axis_name="c", subcore_axis_name="s",
        num_cores=NC, num_subcores=NS,
    )

    @functools.partial(
        pl.kernel,
        out_type=jax.ShapeDtypeStruct((NC, NPAD), jnp.float32),
        mesh=mesh,
        compiler_params=pltpu.CompilerParams(needs_layout_passes=False),
        scratch_types=[
            pltpu.VMEM((NCHUNK, CH), jnp.int32),
            pltpu.VMEM((NCHUNK, CH), jnp.float32),
            pltpu.VMEM((NCHK,), jnp.float32),
            pltpu.VMEM_SHARED((NPAD,), jnp.float32),
        ],
    )
    def deg_kernel(col_b, ew_b, deg_out, eidx, evalf, tmp, deg_sh):
        cid = lax.axis_index("c")
        sid = lax.axis_index("s")
        wid = sid * NC + cid
        nbase = sid * NCHK

        def _z(i, c):
            tmp[pl.ds(i * L, L)] = jnp.zeros((L,), jnp.float32)
            return c
        lax.fori_loop(0, NCHK // L, _z, 0)
        pltpu.sync_copy(tmp, deg_sh.at[pl.ds(nbase, NCHK)])
        plsc.subcore_barrier()

        pltpu.sync_copy(col_b.at[wid], eidx)
        pltpu.sync_copy(ew_b.at[wid], evalf)

        def _deg(j, c):
            pltpu.sync_copy(evalf.at[j], deg_sh.at[eidx.at[j]], add=True)
            return c
        lax.fori_loop(0, NCHUNK, _deg, 0)
        plsc.subcore_barrier()
        pltpu.sync_copy(deg_sh.at[pl.ds(nbase, NCHK)],
                        deg_out.at[cid, pl.ds(nbase, NCHK)])

    @functools.partial(
        pl.kernel,
        out_type=jax.ShapeDtypeStruct((NW, NCHUNK, CH), jnp.float32),
        mesh=mesh,
        compiler_params=pltpu.CompilerParams(needs_layout_passes=False),
        scratch_types=[
            pltpu.VMEM((NCHUNK, CH), jnp.int32),
            pltpu.VMEM((NCHUNK, CH), jnp.int32),
            pltpu.VMEM((NCHUNK, CH), jnp.float32),
            pltpu.VMEM((NCHUNK, CH), jnp.float32),
            pltpu.VMEM((NPAD,), jnp.float32),
        ],
    )
    def norm_kernel(row_b, col_b, ew_b, dinv, norm_out,
                    ridx, cidx, ewb, nbuf, dinv_loc):
        cid = lax.axis_index("c")
        sid = lax.axis_index("s")
        wid = sid * NC + cid

        pltpu.sync_copy(dinv, dinv_loc)
        pltpu.sync_copy(row_b.at[wid], ridx)
        pltpu.sync_copy(col_b.at[wid], cidx)
        pltpu.sync_copy(ew_b.at[wid], ewb)

        def _nrm(j, c):
            for k in range(CH // L):
                sl = pl.ds(k * L, L)
                ri = ridx[j, sl]
                ci = cidx[j, sl]
                w = ewb[j, sl]
                dv = plsc.load_gather(dinv_loc, [ri])
                dc = plsc.load_gather(dinv_loc, [ci])
                nbuf[j, sl] = dv * w * dc
            return c
        lax.fori_loop(0, NCHUNK, _nrm, 0)
        pltpu.sync_copy(nbuf, norm_out.at[wid])

    @functools.partial(
        pl.kernel,
        out_type=jax.ShapeDtypeStruct((NC, NPAD, F), jnp.float32),
        mesh=mesh,
        compiler_params=pltpu.CompilerParams(needs_layout_passes=False),
        scratch_types=[
            pltpu.VMEM((NCHUNK, CH), jnp.int32),
            pltpu.VMEM((2, CH), jnp.int32),
            pltpu.VMEM((2, CH), jnp.float32),
            pltpu.VMEM((CH, F), jnp.float32),
            pltpu.VMEM((CH, F), jnp.float32),
            pltpu.VMEM_SHARED((NPAD, F), jnp.float32),
            pltpu.SemaphoreType.DMA,
            pltpu.SemaphoreType.DMA,
            pltpu.SemaphoreType.DMA,
            pltpu.SemaphoreType.DMA,
            pltpu.SemaphoreType.DMA,
            pltpu.SemaphoreType.DMA,
        ],
    )
    def scatter_kernel(h, row_b, col_b, norm_b, part,
                       ridx, cidxr, nrmr, rows0, rows1, acc_sh,
                       sg0, sg1, si0, si1, ss0, ss1):
        cid = lax.axis_index("c")
        sid = lax.axis_index("s")
        wid = sid * NC + cid
        nbase = sid * NCHK
        rows = (rows0, rows1)
        sg = (sg0, sg1)
        si = (si0, si1)
        ss = (ss0, ss1)

        def _zr(r, c):
            for k in range(F // L):
                rows0[r, pl.ds(k * L, L)] = jnp.zeros((L,), jnp.float32)
            return c
        lax.fori_loop(0, CH, _zr, 0)
        for b in range(NCHK // CH):
            pltpu.sync_copy(rows0, acc_sh.at[pl.ds(nbase + b * CH, CH)])
        plsc.subcore_barrier()

        pltpu.sync_copy(row_b.at[wid], ridx)

        def fire_idx(j, slot):
            pltpu.async_copy(col_b.at[wid, j], cidxr.at[slot], si[slot])
            pltpu.async_copy(norm_b.at[wid, j], nrmr.at[slot], si[slot])

        def wait_idx(slot):
            pltpu.make_async_copy(col_b.at[0, 0], cidxr.at[slot],
                                  si[slot]).wait()
            pltpu.make_async_copy(norm_b.at[0, 0], nrmr.at[slot],
                                  si[slot]).wait()

        def fire_gather(j, slot):
            pltpu.async_copy(h.at[ridx.at[j]], rows[slot], sg[slot])

        def wait_gather(slot):
            pltpu.make_async_copy(h.at[ridx.at[0]], rows[slot],
                                  sg[slot]).wait()

        def fire_scatter(slot):
            pltpu.async_copy(rows[slot], acc_sh.at[cidxr.at[slot]], ss[slot],
                             add=True)

        def wait_scatter(slot):
            pltpu.make_async_copy(rows[slot], acc_sh.at[cidxr.at[slot]],
                                  ss[slot]).wait()

        def scale(slot):
            r = rows[slot]

            def _sce(e, c2):
                se = jnp.full((L,), slot, jnp.int32)
                ee = jnp.full((L,), e, jnp.int32)
                s = plsc.load_gather(nrmr, [se, ee])
                for k in range(F // L):
                    sl = pl.ds(k * L, L)
                    r[e, sl] = r[e, sl] * s
                return c2
            lax.fori_loop(0, CH, _sce, 0, unroll=2)

        def step(j, b, wait_prev, fire_next):
            nb = 1 - b
            wait_gather(b)
            if wait_prev:
                wait_scatter(nb)
            if fire_next:
                fire_idx(j + 1, nb)
                fire_gather(j + 1, nb)
            wait_idx(b)
            scale(b)
            fire_scatter(b)

        fire_idx(0, 0)
        fire_gather(0, 0)
        step(0, 0, wait_prev=False, fire_next=True)
        step(1, 1, wait_prev=True, fire_next=True)

        def _pair(jj, c):
            step(2 * jj, 0, wait_prev=True, fire_next=True)
            step(2 * jj + 1, 1, wait_prev=True, fire_next=True)
            return c
        lax.fori_loop(1, NCHUNK // 2 - 1, _pair, 0)

        step(NCHUNK - 2, 0, wait_prev=True, fire_next=True)
        step(NCHUNK - 1, 1, wait_prev=True, fire_next=False)
        wait_scatter(1)
        plsc.subcore_barrier()

        for b in range(NCHK // CH):
            sl = pl.ds(nbase + b * CH, CH)
            pltpu.sync_copy(acc_sh.at[sl], part.at[cid, sl])

    return deg_kernel, norm_kernel, scatter_kernel


_BLK = 1000


def _dinv_body(dp_ref, dinv_ref, sn_ref):
    d = dp_ref[0] + dp_ref[1] + 1.0
    r = lax.rsqrt(d)
    dinv_ref[...] = r
    sn_ref[...] = r * r


def _tc_dinv(deg_parts):
    m = NPAD // F
    return pl.pallas_call(
        _dinv_body,
        out_shape=(
            jax.ShapeDtypeStruct((m, F), jnp.float32),
            jax.ShapeDtypeStruct((m, F), jnp.float32),
        ),
    )(deg_parts)


def _mm_body(x_ref, w_ref, o_ref):
    o_ref[...] = jnp.dot(x_ref[...], w_ref[...],
                         preferred_element_type=jnp.float32)


def _tc_matmul(x, w):
    return pl.pallas_call(
        _mm_body,
        grid=(N // _BLK,),
        in_specs=[
            pl.BlockSpec((_BLK, F), lambda i: (i, 0)),
            pl.BlockSpec((F, F), lambda i: (0, 0)),
        ],
        out_specs=pl.BlockSpec((_BLK, F), lambda i: (i, 0)),
        out_shape=jax.ShapeDtypeStruct((N, F), jnp.float32),
    )(x, w)


def _comb_body(p0_ref, p1_ref, hm_ref, sn_ref, b_ref, g_ref, bt_ref, w2_ref,
               h1_ref, h2m_ref):
    s = p0_ref[...] + p1_ref[...] + hm_ref[...] * sn_ref[...]
    h = jnp.maximum(s + b_ref[...], 0.0)
    h = h * (g_ref[...] * ISQ) + bt_ref[...]
    h1_ref[...] = h
    h2m_ref[...] = jnp.dot(h, w2_ref[...], preferred_element_type=jnp.float32)


def _tc_combine_mm(p0, p1, hm, sn, b, g, bt, w2):
    vec = pl.BlockSpec((1, F), lambda i: (0, 0))
    blk = pl.BlockSpec((_BLK, F), lambda i: (i, 0))
    col = pl.BlockSpec((_BLK, 1), lambda i: (i, 0))
    return pl.pallas_call(
        _comb_body,
        grid=(N // _BLK,),
        in_specs=[blk, blk, blk, col, vec, vec, vec,
                  pl.BlockSpec((F, F), lambda i: (0, 0))],
        out_specs=(blk, blk),
        out_shape=(
            jax.ShapeDtypeStruct((N, F), jnp.float32),
            jax.ShapeDtypeStruct((N, F), jnp.float32),
        ),
    )(p0, p1, hm, sn, b.reshape(1, F), g.reshape(1, F), bt.reshape(1, F), w2)


def _head_body(p0_ref, p1_ref, hm_ref, sn_ref, b_ref, g_ref, bt_ref,
               x_ref, h1_ref,
               f1a_ref, f1b_ref, f1c_ref, fb1_ref, f2w_ref, f2b_ref, o_ref):
    s = p0_ref[...] + p1_ref[...] + hm_ref[...] * sn_ref[...]
    h2 = jnp.maximum(s + b_ref[...], 0.0)
    h2 = h2 * (g_ref[...] * ISQ) + bt_ref[...]
    z = (jnp.dot(x_ref[...], f1a_ref[...], preferred_element_type=jnp.float32)
         + jnp.dot(h1_ref[...], f1b_ref[...],
                   preferred_element_type=jnp.float32)
         + jnp.dot(h2, f1c_ref[...], preferred_element_type=jnp.float32))
    z = jnp.maximum(z + fb1_ref[...], 0.0)
    y = jnp.dot(z, f2w_ref[...], preferred_element_type=jnp.float32)
    o_ref[...] = jnp.maximum(y + f2b_ref[...], 0.0)


def _tc_head(p0, p1, hm, sn, b2, g2, bt2, x, h1, f1a, f1b, f1c, fb1,
             f2w, f2b):
    vec = pl.BlockSpec((1, F), lambda i: (0, 0))
    blk = pl.BlockSpec((_BLK, F), lambda i: (i, 0))
    col = pl.BlockSpec((_BLK, 1), lambda i: (i, 0))
    wspec = pl.BlockSpec((F, F), lambda i: (0, 0))
    return pl.pallas_call(
        _head_body,
        grid=(N // _BLK,),
        in_specs=[blk, blk, blk, col, vec, vec, vec, blk, blk,
                  wspec, wspec, wspec,
                  vec, pl.BlockSpec((F, 1), lambda i: (0, 0)),
                  pl.BlockSpec((1, 1), lambda i: (0, 0))],
        out_specs=pl.BlockSpec((_BLK, 1), lambda i: (i, 0)),
        out_shape=jax.ShapeDtypeStruct((N, 1), jnp.float32),
    )(p0, p1, hm, sn, b2.reshape(1, F), g2.reshape(1, F), bt2.reshape(1, F),
      x, h1, f1a, f1b, f1c, fb1.reshape(1, F), f2w, f2b.reshape(1, 1))


def kernel(x, adj_indices, adj_values, W1, b1, bn1_gamma, bn1_beta,
           W2, b2, bn2_gamma, bn2_beta, fc1_W, fc1_b, fc2_W, fc2_b):
    deg_kernel, norm_kernel, scatter_kernel = _sc_kernels()
    ipad = jnp.zeros((EP - E,), jnp.int32)
    row_b = jnp.concatenate([adj_indices[0], ipad]).reshape(NW, NCHUNK, CH)
    col_b = jnp.concatenate([adj_indices[1], ipad]).reshape(NW, NCHUNK, CH)
    ew_b = jnp.concatenate([adj_values, jnp.zeros((EP - E,), jnp.float32)]
                           ).reshape(NW, NCHUNK, CH)

    deg_parts = deg_kernel(col_b, ew_b)
    dinv2d, sn2d = _tc_dinv(deg_parts.reshape(NC, NPAD // F, F))
    norm_b = norm_kernel(row_b, col_b, ew_b, dinv2d.reshape(NPAD))
    sn = sn2d.reshape(NPAD)[:N].reshape(N, 1)

    h1m = _tc_matmul(x, W1)
    part1 = scatter_kernel(h1m, row_b, col_b, norm_b)
    h1, h2m = _tc_combine_mm(part1[0, :N], part1[1, :N], h1m, sn,
                             b1, bn1_gamma, bn1_beta, W2)
    part2 = scatter_kernel(h2m, row_b, col_b, norm_b)
    y = _tc_head(part2[0, :N], part2[1, :N], h2m, sn, b2, bn2_gamma, bn2_beta,
                 x, h1, fc1_W[:F], fc1_W[F:2 * F], fc1_W[2 * F:],
                 fc1_b, fc2_W, fc2_b)
    return y.reshape(-1)
